# baseline (speedup 1.0000x reference)
import jax
import jax.numpy as jnp
from jax.experimental import pallas as pl
from jax.experimental.pallas import tpu as pltpu

_NUM_EXPERTS = 8
_HIDDEN = 768
_TILE = 256


def _layer_body(x_ref, rwt_ref, rb_ref, wt_ref, b_ref, out_ref):
    x = x_ref[...]
    logits = jnp.dot(x, rwt_ref[...], preferred_element_type=jnp.float32)
    logits = logits + rb_ref[...]
    e_idx = jax.lax.broadcasted_iota(jnp.int32, (1, _NUM_EXPERTS), 1)
    rank = jnp.zeros(logits.shape, jnp.int32)
    for ep in range(_NUM_EXPERTS):
        lp = logits[:, ep:ep + 1]
        beats = (lp > logits) | ((lp == logits) & (ep < e_idx))
        rank = rank + beats.astype(jnp.int32)
    mask = (rank < 2).astype(jnp.float32)

    acc = jnp.zeros((x.shape[0], _HIDDEN), jnp.float32)
    for e in range(_NUM_EXPERTS):
        ye = jnp.dot(x, wt_ref[e], preferred_element_type=jnp.float32)
        ye = ye + b_ref[e:e + 1, :]
        acc = acc + mask[:, e:e + 1] * ye
    out_ref[...] = acc


def _moe_layer(x, rwt, rb, wt, b):
    n = x.shape[0]
    grid = (n // _TILE,)
    return pl.pallas_call(
        _layer_body,
        grid=grid,
        in_specs=[
            pl.BlockSpec((_TILE, _HIDDEN), lambda i: (i, 0)),
            pl.BlockSpec((_HIDDEN, _NUM_EXPERTS), lambda i: (0, 0)),
            pl.BlockSpec((1, _NUM_EXPERTS), lambda i: (0, 0)),
            pl.BlockSpec((_NUM_EXPERTS, _HIDDEN, _HIDDEN), lambda i: (0, 0, 0)),
            pl.BlockSpec((_NUM_EXPERTS, _HIDDEN), lambda i: (0, 0)),
        ],
        out_specs=pl.BlockSpec((_TILE, _HIDDEN), lambda i: (i, 0)),
        out_shape=jax.ShapeDtypeStruct((n, _HIDDEN), jnp.float32),
    )(x, rwt, rb, wt, b)


def kernel(input_ids, router_w, router_b, expert_w, expert_b):
    batch_size, seq_len = input_ids.shape
    hidden = jax.random.normal(
        jax.random.key(42), (batch_size, seq_len, _HIDDEN), dtype=jnp.float32)
    x = hidden.reshape(batch_size * seq_len, _HIDDEN)
    num_layers = router_w.shape[0]
    for l in range(num_layers):
        rwt = router_w[l].T
        rb = router_b[l].reshape(1, _NUM_EXPERTS)
        wt = jnp.swapaxes(expert_w[l], 1, 2)
        x = _moe_layer(x, rwt, rb, wt, expert_b[l])
    return x.reshape(batch_size, seq_len, _HIDDEN)


# speedup vs baseline: 1.1304x; 1.1304x over previous
import jax
import jax.numpy as jnp
from jax.experimental import pallas as pl
from jax.experimental.pallas import tpu as pltpu

_NUM_EXPERTS = 8
_HIDDEN = 768
_TILE = 256


def _layer_body(x_ref, rwt_ref, rb_ref, wt_ref, b_ref, out_ref):
    x = x_ref[...]
    logits = jnp.dot(x, rwt_ref[...], preferred_element_type=jnp.float32)
    logits = logits + rb_ref[...]
    e_idx = jax.lax.broadcasted_iota(jnp.int32, (1, _NUM_EXPERTS), 1)
    rank = jnp.zeros(logits.shape, jnp.int32)
    for ep in range(_NUM_EXPERTS):
        lp = logits[:, ep:ep + 1]
        beats = (lp > logits) | ((lp == logits) & (ep < e_idx))
        rank = rank + beats.astype(jnp.int32)
    mask = (rank < 2).astype(jnp.float32)

    xb = x.astype(jnp.bfloat16)
    acc = jnp.zeros((x.shape[0], _HIDDEN), jnp.float32)
    for e in range(_NUM_EXPERTS):
        ye = jnp.dot(xb, wt_ref[e], preferred_element_type=jnp.float32)
        ye = ye + b_ref[e:e + 1, :]
        acc = acc + mask[:, e:e + 1] * ye
    out_ref[...] = acc


def _moe_layer(x, rwt, rb, wt, b):
    n = x.shape[0]
    grid = (n // _TILE,)
    return pl.pallas_call(
        _layer_body,
        grid=grid,
        in_specs=[
            pl.BlockSpec((_TILE, _HIDDEN), lambda i: (i, 0)),
            pl.BlockSpec((_HIDDEN, _NUM_EXPERTS), lambda i: (0, 0)),
            pl.BlockSpec((1, _NUM_EXPERTS), lambda i: (0, 0)),
            pl.BlockSpec((_NUM_EXPERTS, _HIDDEN, _HIDDEN), lambda i: (0, 0, 0)),
            pl.BlockSpec((_NUM_EXPERTS, _HIDDEN), lambda i: (0, 0)),
        ],
        compiler_params=pltpu.CompilerParams(
            dimension_semantics=("parallel",),
        ),
        out_specs=pl.BlockSpec((_TILE, _HIDDEN), lambda i: (i, 0)),
        out_shape=jax.ShapeDtypeStruct((n, _HIDDEN), jnp.float32),
    )(x, rwt, rb, wt, b)


def kernel(input_ids, router_w, router_b, expert_w, expert_b):
    batch_size, seq_len = input_ids.shape
    hidden = jax.random.normal(
        jax.random.key(42), (batch_size, seq_len, _HIDDEN), dtype=jnp.float32)
    x = hidden.reshape(batch_size * seq_len, _HIDDEN)
    num_layers = router_w.shape[0]
    for l in range(num_layers):
        rwt = router_w[l].T
        rb = router_b[l].reshape(1, _NUM_EXPERTS)
        wt = jnp.swapaxes(expert_w[l], 1, 2).astype(jnp.bfloat16)
        x = _moe_layer(x, rwt, rb, wt, expert_b[l])
    return x.reshape(batch_size, seq_len, _HIDDEN)
